# baseline (speedup 1.0000x reference)
import functools

import jax
import jax.numpy as jnp
from jax import lax
from jax.experimental import pallas as pl
from jax.experimental.pallas import tpu as pltpu
from jax.experimental.pallas import tpu_sc as plsc


def _conv2d(x, w, b, stride, pad, half=False):
    if half:
        x = x.astype(jnp.bfloat16)
        w = w.astype(jnp.bfloat16)
    y = lax.conv_general_dilated(x, w, (stride, stride), ((pad, pad), (pad, pad)),
                                 dimension_numbers=('NCHW', 'OIHW', 'NCHW'),
                                 preferred_element_type=jnp.float32)
    return y + b[None, :, None, None]


def _conv_t2d_ref(x, w, b, stride, pad):
    assert stride == 2 and pad == 1
    N, C, H, W = x.shape
    xd = jnp.zeros((N, C, (H - 1) * 2 + 1, (W - 1) * 2 + 1), x.dtype)
    xd = xd.at[:, :, ::2, ::2].set(x)
    xe = xd[:, :, ::2, ::2]
    return _conv_t2d(xe, w, b, 2, 1)


def _conv_t2d(x, w, b, stride, pad):
    assert stride == 2 and w.shape[2] == 4 and w.shape[3] == 4 and pad == 1
    w2 = jnp.flip(w, axis=(2, 3)).transpose(1, 0, 2, 3)
    O = w.shape[1]
    N, C, H, W = x.shape
    ks = [w2[:, :, r::2, s::2] for r in (0, 1) for s in (0, 1)]
    kall = jnp.concatenate(ks, axis=0).astype(jnp.bfloat16)
    xp = jnp.pad(x.astype(jnp.bfloat16), ((0, 0), (0, 0), (1, 1), (1, 1)))
    y = lax.conv_general_dilated(xp, kall, (1, 1), ((0, 0), (0, 0)),
                                 dimension_numbers=('NCHW', 'OIHW', 'NCHW'),
                                 preferred_element_type=jnp.float32)
    ys = {}
    for r in (0, 1):
        for s in (0, 1):
            c0 = (2 * r + s) * O
            ys[(r, s)] = y[:, c0:c0 + O, r:r + H, s:s + W]
    t0 = jnp.stack([ys[(0, 0)], ys[(0, 1)]], axis=-1)
    t1 = jnp.stack([ys[(1, 0)], ys[(1, 1)]], axis=-1)
    out = jnp.stack([t0, t1], axis=3)
    out = out.reshape(N, O, 2 * H, 2 * W)
    return out + b[None, :, None, None]


def _batchnorm(x, g, b, eps=1e-5):
    m = jnp.mean(x, axis=(0, 2, 3), keepdims=True)
    v = jnp.var(x, axis=(0, 2, 3), keepdims=True)
    xh = (x - m) / jnp.sqrt(v + eps)
    return xh * g[None, :, None, None] + b[None, :, None, None]


def _vq_argmin_body(qi_ref, cb_ref, idx_ref):
    qi = qi_ref[:]
    cb = cb_ref[:]
    K = cb.shape[0]
    qn = jnp.sum(qi * qi, axis=1, keepdims=True)
    cn = jnp.sum(cb * cb, axis=1)[None, :]
    prod = lax.dot_general(qi, cb, (((1,), (1,)), ((), ())),
                           preferred_element_type=jnp.float32)
    d2 = jnp.maximum(qn + cn - 2.0 * prod, 0.0)
    minval = jnp.min(d2, axis=1, keepdims=True)
    kiota = lax.broadcasted_iota(jnp.int32, d2.shape, 1)
    idx = jnp.min(jnp.where(d2 == minval, kiota, K), axis=1)
    idx_ref[:] = idx[None, :]


_SC_MESH = plsc.VectorSubcoreMesh(core_axis_name="c", subcore_axis_name="s")


def _vq_gather_body(n_valid, idx_hbm, cb_hbm, qi_hbm, quant_hbm, part_hbm,
                    idx_v, rows_v, qi_v, acc_v, sem):
    nc = 2
    wid = lax.axis_index("s") * nc + lax.axis_index("c")
    base = wid * 16
    pltpu.sync_copy(idx_hbm.at[pl.ds(base, 16)], idx_v)
    pltpu.async_copy(cb_hbm.at[idx_v], rows_v, sem).wait()
    pltpu.sync_copy(qi_hbm.at[pl.ds(base, 16)], qi_v)
    acc1 = jnp.zeros((16,), jnp.float32)
    acc2 = jnp.zeros((16,), jnp.float32)
    for i in range(16):
        m = jnp.where(base + i < n_valid, 1.0, 0.0)
        for g in range(8):
            vq = rows_v[i, pl.ds(g * 16, 16)]
            vz = qi_v[i, pl.ds(g * 16, 16)]
            diff = vq - vz
            acc1 = acc1 + m * (diff * diff)
            acc2 = acc2 + m * (vq - vz * vz)
    acc_v[0, :] = acc1
    acc_v[1, :] = acc2
    pltpu.sync_copy(rows_v, quant_hbm.at[pl.ds(base, 16)])
    pltpu.sync_copy(acc_v, part_hbm.at[wid])


def _vq_loss_body(n_valid, part_ref, loss_ref):
    part = part_ref[:]
    denom = jnp.float32(n_valid * 128)
    commitment = jnp.sum(part[:, 0, :]) / denom
    codebook_loss = jnp.sum(part[:, 1, :]) / denom
    loss = codebook_loss + 0.25 * commitment
    loss_ref[:] = jnp.full(loss_ref.shape, loss, jnp.float32)


def _vq_quantize(qi_flat, cb):
    N, C = qi_flat.shape
    K = cb.shape[0]
    NP = 512
    qi_pad = jnp.pad(qi_flat, ((0, NP - N), (0, 0)))
    idx2 = pl.pallas_call(
        _vq_argmin_body,
        out_shape=jax.ShapeDtypeStruct((1, NP), jnp.int32),
    )(qi_pad, cb)
    idx = idx2.reshape(NP)

    sc_kernel = functools.partial(
        pl.kernel,
        mesh=_SC_MESH,
        out_type=[
            jax.ShapeDtypeStruct((NP, C), jnp.float32),
            jax.ShapeDtypeStruct((32, 2, 16), jnp.float32),
        ],
        scratch_types=[
            pltpu.VMEM((16,), jnp.int32),
            pltpu.VMEM((16, C), jnp.float32),
            pltpu.VMEM((16, C), jnp.float32),
            pltpu.VMEM((2, 16), jnp.float32),
            pltpu.SemaphoreType.DMA,
        ],
    )
    quant_pad, part = sc_kernel(functools.partial(_vq_gather_body, N))(
        idx, cb, qi_pad)

    lossbuf = pl.pallas_call(
        functools.partial(_vq_loss_body, N),
        out_shape=jax.ShapeDtypeStruct((8, 128), jnp.float32),
    )(part)
    return quant_pad[:N], lossbuf[0, 0]


def kernel(x, params):
    beta = 0.25
    h = x
    for i in range(5):
        h = _conv2d(h, params[f'enc_w{i}'], params[f'enc_b{i}'], 2, 1)
        h = _batchnorm(h, params[f'enc_g{i}'], params[f'enc_be{i}'])
        h = jax.nn.relu(h)
    qi = _conv2d(h, params['pre_w'], params['pre_b'], 1, 0)
    B, C, H, W = qi.shape
    qi_flat = qi.transpose(0, 2, 3, 1).reshape(-1, C)

    quant, quantize_losses = _vq_quantize(qi_flat, params['codebook'])

    quant = quant.reshape(B, H, W, C).transpose(0, 3, 1, 2)
    d = _conv2d(quant, params['post_w'], params['post_b'], 1, 0, half=True)
    for i in range(5):
        t2d = _conv_t2d if i < 4 else _conv_t2d_ref
        d = t2d(d, params[f'dec_w{i}'], params[f'dec_b{i}'], 2, 1)
        d = _batchnorm(d, params[f'dec_g{i}'], params[f'dec_be{i}'])
        if i < 4:
            d = jax.nn.relu(d)
        else:
            d = jax.nn.sigmoid(d)
    return d, quantize_losses


# speedup vs baseline: 1.1335x; 1.1335x over previous
import functools

import jax
import jax.numpy as jnp
from jax import lax
from jax.experimental import pallas as pl
from jax.experimental.pallas import tpu as pltpu
from jax.experimental.pallas import tpu_sc as plsc


def _conv2d(x, w, b, stride, pad):
    y = lax.conv_general_dilated(x, w, (stride, stride), ((pad, pad), (pad, pad)),
                                 dimension_numbers=('NCHW', 'OIHW', 'NCHW'))
    return y + b[None, :, None, None]


def _conv_t2d_ref(x, w, b, stride, pad):
    assert stride == 2 and pad == 1
    N, C, H, W = x.shape
    xd = jnp.zeros((N, C, (H - 1) * 2 + 1, (W - 1) * 2 + 1), x.dtype)
    xd = xd.at[:, :, ::2, ::2].set(x)
    xe = xd[:, :, ::2, ::2]
    return _conv_t2d(xe, w, b, 2, 1)


def _conv_t2d(x, w, b, stride, pad):
    assert stride == 2 and w.shape[2] == 4 and w.shape[3] == 4 and pad == 1
    w2 = jnp.flip(w, axis=(2, 3)).transpose(1, 0, 2, 3)
    O = w.shape[1]
    N, C, H, W = x.shape
    ks = [w2[:, :, r::2, s::2] for r in (0, 1) for s in (0, 1)]
    kall = jnp.concatenate(ks, axis=0)
    xp = jnp.pad(x, ((0, 0), (0, 0), (1, 1), (1, 1)))
    y = lax.conv_general_dilated(xp, kall, (1, 1), ((0, 0), (0, 0)),
                                 dimension_numbers=('NCHW', 'OIHW', 'NCHW'))
    ys = {}
    for r in (0, 1):
        for s in (0, 1):
            c0 = (2 * r + s) * O
            ys[(r, s)] = y[:, c0:c0 + O, r:r + H, s:s + W]
    t0 = jnp.stack([ys[(0, 0)], ys[(0, 1)]], axis=-1)
    t1 = jnp.stack([ys[(1, 0)], ys[(1, 1)]], axis=-1)
    out = jnp.stack([t0, t1], axis=3)
    out = out.reshape(N, O, 2 * H, 2 * W)
    return out + b[None, :, None, None]


def _batchnorm(x, g, b, eps=1e-5):
    m = jnp.mean(x, axis=(0, 2, 3), keepdims=True)
    ms = jnp.mean(x * x, axis=(0, 2, 3), keepdims=True)
    v = jnp.maximum(ms - m * m, 0.0)
    scale = g[None, :, None, None] / jnp.sqrt(v + eps)
    return (x - m) * scale + b[None, :, None, None]


def _vq_argmin_body(qi_ref, cb_ref, idx_ref):
    qi = qi_ref[:]
    cb = cb_ref[:]
    K = cb.shape[0]
    qn = jnp.sum(qi * qi, axis=1, keepdims=True)
    cn = jnp.sum(cb * cb, axis=1)[None, :]
    prod = lax.dot_general(qi, cb, (((1,), (1,)), ((), ())),
                           preferred_element_type=jnp.float32)
    d2 = jnp.maximum(qn + cn - 2.0 * prod, 0.0)
    minval = jnp.min(d2, axis=1, keepdims=True)
    kiota = lax.broadcasted_iota(jnp.int32, d2.shape, 1)
    idx = jnp.min(jnp.where(d2 == minval, kiota, K), axis=1)
    idx_ref[:] = idx[None, :]


_SC_MESH = plsc.VectorSubcoreMesh(core_axis_name="c", subcore_axis_name="s")


def _vq_gather_body(n_valid, idx_hbm, cb_hbm, qi_hbm, quant_hbm, part_hbm,
                    idx_v, rows_v, qi_v, acc_v, sem):
    nc = 2
    wid = lax.axis_index("s") * nc + lax.axis_index("c")
    base = wid * 16
    pltpu.sync_copy(idx_hbm.at[pl.ds(base, 16)], idx_v)
    pltpu.async_copy(cb_hbm.at[idx_v], rows_v, sem).wait()
    pltpu.sync_copy(qi_hbm.at[pl.ds(base, 16)], qi_v)
    acc1 = jnp.zeros((16,), jnp.float32)
    acc2 = jnp.zeros((16,), jnp.float32)
    for i in range(16):
        m = jnp.where(base + i < n_valid, 1.0, 0.0)
        for g in range(8):
            vq = rows_v[i, pl.ds(g * 16, 16)]
            vz = qi_v[i, pl.ds(g * 16, 16)]
            diff = vq - vz
            acc1 = acc1 + m * (diff * diff)
            acc2 = acc2 + m * (vq - vz * vz)
    acc_v[0, :] = acc1
    acc_v[1, :] = acc2
    pltpu.sync_copy(rows_v, quant_hbm.at[pl.ds(base, 16)])
    pltpu.sync_copy(acc_v, part_hbm.at[wid])


def _vq_loss_body(n_valid, part_ref, loss_ref):
    part = part_ref[:]
    denom = jnp.float32(n_valid * 128)
    commitment = jnp.sum(part[:, 0, :]) / denom
    codebook_loss = jnp.sum(part[:, 1, :]) / denom
    loss = codebook_loss + 0.25 * commitment
    loss_ref[:] = jnp.full(loss_ref.shape, loss, jnp.float32)


def _vq_quantize(qi_flat, cb):
    N, C = qi_flat.shape
    K = cb.shape[0]
    NP = 512
    qi_pad = jnp.pad(qi_flat, ((0, NP - N), (0, 0)))
    idx2 = pl.pallas_call(
        _vq_argmin_body,
        out_shape=jax.ShapeDtypeStruct((1, NP), jnp.int32),
    )(qi_pad, cb)
    idx = idx2.reshape(NP)

    sc_kernel = functools.partial(
        pl.kernel,
        mesh=_SC_MESH,
        out_type=[
            jax.ShapeDtypeStruct((NP, C), jnp.float32),
            jax.ShapeDtypeStruct((32, 2, 16), jnp.float32),
        ],
        scratch_types=[
            pltpu.VMEM((16,), jnp.int32),
            pltpu.VMEM((16, C), jnp.float32),
            pltpu.VMEM((16, C), jnp.float32),
            pltpu.VMEM((2, 16), jnp.float32),
            pltpu.SemaphoreType.DMA,
        ],
    )
    quant_pad, part = sc_kernel(functools.partial(_vq_gather_body, N))(
        idx, cb, qi_pad)

    lossbuf = pl.pallas_call(
        functools.partial(_vq_loss_body, N),
        out_shape=jax.ShapeDtypeStruct((8, 128), jnp.float32),
    )(part)
    return quant_pad[:N], lossbuf[0, 0]


def kernel(x, params):
    beta = 0.25
    h = x
    for i in range(5):
        h = _conv2d(h, params[f'enc_w{i}'], params[f'enc_b{i}'], 2, 1)
        h = _batchnorm(h, params[f'enc_g{i}'], params[f'enc_be{i}'])
        h = jax.nn.relu(h)
    qi = _conv2d(h, params['pre_w'], params['pre_b'], 1, 0)
    B, C, H, W = qi.shape
    qi_flat = qi.transpose(0, 2, 3, 1).reshape(-1, C)

    quant, quantize_losses = _vq_quantize(qi_flat, params['codebook'])

    quant = quant.reshape(B, H, W, C).transpose(0, 3, 1, 2)
    d = _conv2d(quant, params['post_w'], params['post_b'], 1, 0)
    for i in range(5):
        t2d = _conv_t2d if i < 4 else _conv_t2d_ref
        d = t2d(d, params[f'dec_w{i}'], params[f'dec_b{i}'], 2, 1)
        d = _batchnorm(d, params[f'dec_g{i}'], params[f'dec_be{i}'])
        if i < 4:
            d = jax.nn.relu(d)
        else:
            d = jax.nn.sigmoid(d)
    return d, quantize_losses
